# Initial kernel scaffold; baseline (speedup 1.0000x reference)
import jax
import jax.numpy as jnp
from jax.experimental import pallas as pl


def kernel(x, edge_index, W1, a_src1, a_dst1, b1, ln1_g, ln1_b, W2, a_src2, a_dst2, b2, ln2_g, ln2_b, W3, a_src3, a_dst3, b3, lno_g, lno_b):
    raise NotImplementedError("write your pallas kernel here")


# speedup vs baseline: 10.2914x; 10.2914x over previous
import jax
import jax.numpy as jnp
from jax import lax
from jax.experimental import pallas as pl
from jax.experimental.pallas import tpu as pltpu
from jax.experimental.pallas import tpu_sc as plsc

N = 10000
E = 160000
NT = 16
NB = 80
BLK = 128
EP = NT * NB * BLK
NPAD = 10240
RPT = NPAD // NT
NEG = 0.2
LN = 16


RB = 128
GRID = (N + RB - 1) // RB


def _full_spec(shape):
    return pl.BlockSpec(shape, lambda i: (0,) * len(shape))


def _row_spec(cols):
    return pl.BlockSpec((RB, cols), lambda i: (i, 0))


def _dense_first(x, W, Asrc, Adst):
    Fin, Fout = W.shape

    def body(x_ref, w_ref, as_ref, ad_ref, xp_ref, ss_ref, sd_ref):
        xp = jnp.dot(x_ref[...], w_ref[...], preferred_element_type=jnp.float32)
        xp_ref[...] = xp
        ss_ref[...] = jnp.dot(xp, as_ref[...], preferred_element_type=jnp.float32)
        sd_ref[...] = jnp.dot(xp, ad_ref[...], preferred_element_type=jnp.float32)

    return pl.pallas_call(
        body,
        grid=(GRID,),
        in_specs=[_row_spec(Fin), _full_spec((Fin, Fout)),
                  _full_spec((Fout, LN)), _full_spec((Fout, LN))],
        out_specs=[_row_spec(Fout), _row_spec(LN), _row_spec(LN)],
        out_shape=[
            jax.ShapeDtypeStruct((N, Fout), jnp.float32),
            jax.ShapeDtypeStruct((N, LN), jnp.float32),
            jax.ShapeDtypeStruct((N, LN), jnp.float32),
        ],
    )(x, W, Asrc, Adst)


def _dense_mid(planes, b, g, beta, W, Asrc, Adst):
    P = len(planes)
    Cp = planes[0].shape[1]
    Fin = P * Cp
    Fout = W.shape[1]

    def body(*refs):
        plane_refs = refs[:P]
        b_ref, g_ref, beta_ref, w_ref, as_ref, ad_ref = refs[P:P + 6]
        xp_ref, ss_ref, sd_ref = refs[P + 6:]
        h = jnp.concatenate([r[...] for r in plane_refs], axis=1) + b_ref[...]
        mu = jnp.mean(h, axis=-1, keepdims=True)
        var = jnp.mean(jnp.square(h - mu), axis=-1, keepdims=True)
        h = (h - mu) / jnp.sqrt(var + 1e-5) * g_ref[...] + beta_ref[...]
        h = jnp.where(h > 0, h, jnp.exp(jnp.minimum(h, 0.0)) - 1.0)
        xp = jnp.dot(h, w_ref[...], preferred_element_type=jnp.float32)
        xp_ref[...] = xp
        ss_ref[...] = jnp.dot(xp, as_ref[...], preferred_element_type=jnp.float32)
        sd_ref[...] = jnp.dot(xp, ad_ref[...], preferred_element_type=jnp.float32)

    return pl.pallas_call(
        body,
        grid=(GRID,),
        in_specs=[_row_spec(Cp)] * P
        + [_full_spec((1, Fin))] * 3
        + [_full_spec((Fin, Fout)), _full_spec((Fout, LN)),
           _full_spec((Fout, LN))],
        out_specs=[_row_spec(Fout), _row_spec(LN), _row_spec(LN)],
        out_shape=[
            jax.ShapeDtypeStruct((N, Fout), jnp.float32),
            jax.ShapeDtypeStruct((N, LN), jnp.float32),
            jax.ShapeDtypeStruct((N, LN), jnp.float32),
        ],
    )(*planes, b, g, beta, W, Asrc, Adst)


def _dense_final(p0, p1, b, g, beta):
    F = p0.shape[1]

    def body(p0_ref, p1_ref, b_ref, g_ref, beta_ref, o_ref):
        h = p0_ref[...] + p1_ref[...] + b_ref[...]
        mu = jnp.mean(h, axis=-1, keepdims=True)
        var = jnp.mean(jnp.square(h - mu), axis=-1, keepdims=True)
        o_ref[...] = (h - mu) / jnp.sqrt(var + 1e-5) * g_ref[...] + beta_ref[...]

    return pl.pallas_call(
        body,
        grid=(GRID,),
        in_specs=[_row_spec(F), _row_spec(F)] + [_full_spec((1, F))] * 3,
        out_specs=_row_spec(F),
        out_shape=jax.ShapeDtypeStruct((N, F), jnp.float32),
    )(p0, p1, b, g, beta)


def _make_sc_layer(H, WROW, layer_id):
    NV = WROW // 16
    npasses = 4 if layer_id == 1 else 1
    P = 8 if layer_id == 1 else 2
    mesh = plsc.VectorSubcoreMesh(core_axis_name="c", subcore_axis_name="s")

    out_type = [
        jax.ShapeDtypeStruct((P, N, WROW), jnp.float32),
        jax.ShapeDtypeStruct((2, NT, NB, BLK, LN), jnp.float32),
        jax.ShapeDtypeStruct((2, NPAD, LN), jnp.float32),
    ]

    scratch = [
        pltpu.VMEM((NB, BLK), jnp.int32),
        pltpu.VMEM((NB, BLK), jnp.int32),
        pltpu.VMEM((BLK, WROW), jnp.float32),
        pltpu.VMEM((BLK, LN), jnp.float32),
        pltpu.VMEM((BLK, LN), jnp.float32),
        pltpu.VMEM((2048,), jnp.float32),
        pltpu.VMEM((NT * 16,), jnp.float32),
        pltpu.VMEM_SHARED((NPAD, LN), jnp.float32),
        pltpu.VMEM_SHARED((NT * 16,), jnp.float32),
        pltpu.VMEM_SHARED((NPAD, WROW), jnp.float32),
    ]

    def body(src_hbm, dst_hbm, ssrc_hbm, sdst_hbm, tabs_hbm,
             out_hbm, ax_hbm, den_hbm,
             src_v, dst_v, rows, sbuf, dbuf, zbuf, gbuf,
             den_sh, gmax_sh, acc_sh):
        c = lax.axis_index("c")
        s = lax.axis_index("s")
        iota = lax.iota(jnp.int32, 16)

        pltpu.sync_copy(src_hbm.at[s], src_v)
        pltpu.sync_copy(dst_hbm.at[s], dst_v)

        def zsb(r, _):
            sbuf[r, pl.ds(0, 16)] = jnp.zeros((16,), jnp.float32)
            return 0

        lax.fori_loop(0, BLK, zsb, 0)
        for ch in range(NPAD // BLK):
            @pl.when(s == (ch % NT))
            def _():
                pltpu.sync_copy(sbuf, den_sh.at[pl.ds(ch * BLK, BLK)])

        def a1_body(j, gm):
            pltpu.sync_copy(ssrc_hbm.at[src_v.at[j]], sbuf)
            pltpu.sync_copy(sdst_hbm.at[dst_v.at[j]], dbuf)

            def erow(r, gm):
                v = sbuf[r, pl.ds(0, 16)] + dbuf[r, pl.ds(0, 16)]
                e = jnp.maximum(v, NEG * v)
                sbuf[r, pl.ds(0, 16)] = e
                return jnp.maximum(gm, e)

            gm = lax.fori_loop(0, BLK, erow, gm)
            pltpu.sync_copy(sbuf, ax_hbm.at[c, s, j])
            return gm

        gm = lax.fori_loop(0, NB, a1_body, jnp.full((16,), -1e30, jnp.float32))

        zbuf[pl.ds(0, 16)] = gm
        pltpu.sync_copy(zbuf.at[pl.ds(0, 16)], gmax_sh.at[pl.ds(s * 16, 16)])
        plsc.subcore_barrier()
        pltpu.sync_copy(gmax_sh, gbuf)
        gacc = gbuf[pl.ds(0, 16)]
        for t in range(1, NT):
            gacc = jnp.maximum(gacc, gbuf[pl.ds(t * 16, 16)])
        gs = gacc[0]
        for l in range(1, 16):
            gs = jnp.maximum(gs, gacc[l])
        g = jnp.full((16,), gs)

        def a2_body(j, _):
            pltpu.sync_copy(ax_hbm.at[c, s, j], sbuf)
            ebase = s * (NB * BLK) + j * BLK

            hmask = (-((iota - H) >> 31)).astype(jnp.float32)

            def xrow(r, _):
                e = sbuf[r, pl.ds(0, 16)]
                live = jnp.where(ebase + r < E, 1.0, 0.0)
                exv = jnp.exp(e - g) * hmask * jnp.full((16,), live)
                sbuf[r, pl.ds(0, 16)] = exv
                return 0

            lax.fori_loop(0, BLK, xrow, 0)
            pltpu.sync_copy(sbuf, ax_hbm.at[c, s, j])
            pltpu.sync_copy(sbuf, den_sh.at[dst_v.at[j]], add=True)
            return 0

        lax.fori_loop(0, NB, a2_body, 0)
        plsc.subcore_barrier()
        pltpu.sync_copy(den_sh.at[pl.ds(s * RPT, RPT)],
                        den_hbm.at[c, pl.ds(s * RPT, RPT)])
        plsc.subcore_barrier()

        def a3_body(j, _):
            pltpu.sync_copy(ax_hbm.at[c, s, j], sbuf)
            pltpu.sync_copy(den_hbm.at[c].at[dst_v.at[j]], dbuf)

            def arow(r, _):
                ex = sbuf[r, pl.ds(0, 16)]
                den = dbuf[r, pl.ds(0, 16)]
                sbuf[r, pl.ds(0, 16)] = ex / (den + 1e-16)
                return 0

            lax.fori_loop(0, BLK, arow, 0)
            pltpu.sync_copy(sbuf, ax_hbm.at[c, s, j])
            return 0

        lax.fori_loop(0, NB, a3_body, 0)

        for p in range(npasses):
            def zrows(r, _):
                for t in range(NV):
                    rows[r, pl.ds(16 * t, 16)] = jnp.zeros((16,), jnp.float32)
                return 0

            lax.fori_loop(0, BLK, zrows, 0)
            for chn in range(RPT // BLK):
                pltpu.sync_copy(rows,
                                acc_sh.at[pl.ds(s * RPT + chn * BLK, BLK)])
            plsc.subcore_barrier()

            if layer_id == 1:
                tidx = 4 * c + p
                plane = 4 * c + p
                jlo, jhi = 0, NB
            elif layer_id == 2:
                tidx = c
                plane = c
                jlo, jhi = 0, NB
            else:
                tidx = 0
                plane = c
                jlo, jhi = 40 * c, 40 * c + 40

            def b_body(j, _, p=p):
                pltpu.sync_copy(tabs_hbm.at[tidx].at[src_v.at[j]], rows)
                pltpu.sync_copy(ax_hbm.at[c, s, j], sbuf)

                def wrow(r, _, p=p):
                    av = sbuf[r, pl.ds(0, 16)]
                    if layer_id == 1:
                        a = jnp.where(c == 0, av[p // 2], av[2 + p // 2])
                        m = jnp.full((16,), a)
                        for t in range(NV):
                            rows[r, pl.ds(16 * t, 16)] = (
                                rows[r, pl.ds(16 * t, 16)] * m)
                    elif layer_id == 2:
                        a0 = jnp.where(c == 0, av[0], av[2])
                        a1 = jnp.where(c == 0, av[1], av[3])
                        m0 = jnp.full((16,), a0)
                        m1 = jnp.full((16,), a1)
                        mm = jnp.where(iota < 8, m0, m1)
                        for t, mv in enumerate([m0, m0, mm, m1, m1]):
                            rows[r, pl.ds(16 * t, 16)] = (
                                rows[r, pl.ds(16 * t, 16)] * mv)
                    else:
                        m = jnp.full((16,), av[0])
                        for t in range(NV):
                            rows[r, pl.ds(16 * t, 16)] = (
                                rows[r, pl.ds(16 * t, 16)] * m)
                    return 0

                lax.fori_loop(0, BLK, wrow, 0)
                pltpu.sync_copy(rows, acc_sh.at[dst_v.at[j]], add=True)
                return 0

            lax.fori_loop(jlo, jhi, b_body, 0)
            plsc.subcore_barrier()

            @pl.when(s < NT - 1)
            def _():
                pltpu.sync_copy(acc_sh.at[pl.ds(s * RPT, RPT)],
                                out_hbm.at[plane, pl.ds(s * RPT, RPT)])

            @pl.when(s == NT - 1)
            def _():
                tail = N - (NT - 1) * RPT
                pltpu.sync_copy(
                    acc_sh.at[pl.ds((NT - 1) * RPT, tail)],
                    out_hbm.at[plane, pl.ds((NT - 1) * RPT, tail)])

    return pl.kernel(
        body, out_type=out_type, mesh=mesh, scratch_types=scratch,
        compiler_params=pltpu.CompilerParams(use_tc_tiling_on_sc=False),
        name=f"gat_sc_layer{layer_id}")


_sc1 = _make_sc_layer(4, 64, 1)
_sc2 = _make_sc_layer(4, 80, 2)
_sc3 = _make_sc_layer(1, 48, 3)


def _build_a16(a):
    H, C = a.shape
    z = jnp.zeros((H, C, LN), jnp.float32)
    idx = jnp.arange(H)
    z = z.at[idx, :, idx].set(a)
    return z.reshape(H * C, LN)


def kernel(x, edge_index, W1, a_src1, a_dst1, b1, ln1_g, ln1_b,
           W2, a_src2, a_dst2, b2, ln2_g, ln2_b,
           W3, a_src3, a_dst3, b3, lno_g, lno_b):
    src = edge_index[0]
    dst = edge_index[1]
    pad_idx = (jnp.arange(E, EP, dtype=jnp.int32) % N).astype(jnp.int32)
    src_p = jnp.concatenate([src, pad_idx]).reshape(NT, NB, BLK)
    dst_p = jnp.concatenate([dst, pad_idx]).reshape(NT, NB, BLK)

    xp1, ss1, sd1 = _dense_first(x, W1, _build_a16(a_src1), _build_a16(a_dst1))
    tabs1 = xp1.reshape(N, 8, 64).transpose(1, 0, 2)
    out1, _ax1, _den1 = _sc1(src_p, dst_p, ss1, sd1, tabs1)

    xp2, ss2, sd2 = _dense_mid(
        [out1[i] for i in range(8)],
        b1.reshape(1, -1), ln1_g.reshape(1, -1), ln1_b.reshape(1, -1),
        W2, _build_a16(a_src2), _build_a16(a_dst2))
    tabs2 = xp2.reshape(N, 2, 80).transpose(1, 0, 2)
    out2, _ax2, _den2 = _sc2(src_p, dst_p, ss2, sd2, tabs2)

    W3p = jnp.pad(W3, ((0, 0), (0, 8)))
    A3s = jnp.pad(_build_a16(a_src3), ((0, 8), (0, 0)))
    A3d = jnp.pad(_build_a16(a_dst3), ((0, 8), (0, 0)))
    xp3p, ss3, sd3 = _dense_mid(
        [out2[0], out2[1]],
        b2.reshape(1, -1), ln2_g.reshape(1, -1), ln2_b.reshape(1, -1),
        W3p, A3s, A3d)
    tabs3 = xp3p.reshape(N, 1, 48).transpose(1, 0, 2)
    out3, ax3, _den3 = _sc3(src_p, dst_p, ss3, sd3, tabs3)

    out = _dense_final(out3[0, :, :40], out3[1, :, :40],
                       b3.reshape(1, -1), lno_g.reshape(1, -1),
                       lno_b.reshape(1, -1))

    h1 = xp1.reshape(N, 4, 128)
    h2 = xp2.reshape(N, 4, 40)
    h3 = xp3p[:, :40].reshape(N, 1, 40)
    alpha_o = ax3[0, :, :, :, 0].reshape(EP)[:E].reshape(E, 1)
    return (out, h1, h2, h3, alpha_o)
